# Initial kernel scaffold; baseline (speedup 1.0000x reference)
import jax
import jax.numpy as jnp
from jax.experimental import pallas as pl


def kernel(x, W, b):
    raise NotImplementedError("write your pallas kernel here")


# speedup vs baseline: 5.7293x; 5.7293x over previous
import functools

import jax
import jax.numpy as jnp
from jax.experimental import pallas as pl
from jax.experimental.pallas import tpu as pltpu

HIDDEN = 1024
EXPERTS = 64
TOPK = 8
TOKENS = 32768
BLOCK = 512


def _gating_kernel(x_ref, w_ref, b_ref, sparse_ref, idx_ref, logits_ref):
    x = x_ref[...]
    w = w_ref[...]
    logits = jax.lax.dot_general(
        x, w, (((1,), (1,)), ((), ())), preferred_element_type=jnp.float32
    )
    logits = logits + b_ref[...]
    logits_ref[...] = logits

    rows = logits.shape[0]
    col = jax.lax.broadcasted_iota(jnp.int32, (rows, EXPERTS), 1)
    work = logits
    sel = jnp.zeros((rows, EXPERTS), jnp.bool_)
    idx_cols = []
    for _ in range(TOPK):
        m = jnp.max(work, axis=1, keepdims=True)
        is_m = work == m
        amin = jnp.min(jnp.where(is_m, col, EXPERTS), axis=1, keepdims=True)
        idx_cols.append(amin)
        hit = col == amin
        sel = jnp.logical_or(sel, hit)
        work = jnp.where(hit, -jnp.inf, work)
    idx_ref[...] = jnp.concatenate(idx_cols, axis=1)

    m0 = jnp.max(logits, axis=1, keepdims=True)
    e = jnp.where(sel, jnp.exp(logits - m0), 0.0)
    s = jnp.sum(e, axis=1, keepdims=True)
    sparse_ref[...] = e / s


@jax.jit
def kernel(x, W, b):
    b2 = b.reshape(1, EXPERTS)
    grid = (TOKENS // BLOCK,)
    sparse, idx, logits = pl.pallas_call(
        _gating_kernel,
        grid=grid,
        in_specs=[
            pl.BlockSpec((BLOCK, HIDDEN), lambda i: (i, 0)),
            pl.BlockSpec((EXPERTS, HIDDEN), lambda i: (0, 0)),
            pl.BlockSpec((1, EXPERTS), lambda i: (0, 0)),
        ],
        out_specs=[
            pl.BlockSpec((BLOCK, EXPERTS), lambda i: (i, 0)),
            pl.BlockSpec((BLOCK, TOPK), lambda i: (i, 0)),
            pl.BlockSpec((BLOCK, EXPERTS), lambda i: (i, 0)),
        ],
        out_shape=[
            jax.ShapeDtypeStruct((TOKENS, EXPERTS), jnp.float32),
            jax.ShapeDtypeStruct((TOKENS, TOPK), jnp.int32),
            jax.ShapeDtypeStruct((TOKENS, EXPERTS), jnp.float32),
        ],
        compiler_params=pltpu.CompilerParams(
            dimension_semantics=("arbitrary",),
        ),
    )(x, W, b2)
    return sparse, idx, logits
